# baseline (speedup 1.0000x reference)
import functools

import jax
import jax.numpy as jnp
from jax import lax
from jax.experimental import pallas as pl
from jax.experimental.pallas import tpu as pltpu
from jax.experimental.pallas import tpu_sc as plsc

_BEAM = 8
_VOCAB = 100000
_NR = 512
_VB = 1024
_NB = -(-_VOCAB // _VB)
_NSTR = _VB // _BEAM
_END = 2
_MINLEN = 10
_NEG = -1e30
_BIGI = 1 << 30
_GW = 128


@functools.cache
def _make_sc_gather(nrows):
    info = plsc.get_sparse_core_info()
    nw = info.num_cores * info.num_subcores
    bpw = nrows // nw
    mesh = plsc.VectorSubcoreMesh(core_axis_name="c", subcore_axis_name="s")

    @functools.partial(
        pl.kernel,
        out_type=jax.ShapeDtypeStruct((nrows, _GW), jnp.int32),
        mesh=mesh,
        scratch_types=[
            pltpu.VMEM((bpw,), jnp.int32),
            pltpu.VMEM((bpw, _GW), jnp.int32),
            pltpu.SemaphoreType.DMA,
        ],
    )
    def sc_gather(gb_hbm, idx_hbm, out_hbm, idx_v, rows_v, sem):
        wid = lax.axis_index("s") * info.num_cores + lax.axis_index("c")
        base = wid * bpw
        pltpu.sync_copy(idx_hbm.at[pl.ds(base, bpw)], idx_v)
        pltpu.async_copy(gb_hbm.at[idx_v], rows_v, sem).wait()
        pltpu.sync_copy(rows_v, out_hbm.at[pl.ds(base, bpw)])

    return sc_gather


def _merge_top2(av1, ap1, av2, ap2, bv1, bp1, bv2, bp2):
    swap = (bv1 > av1) | ((bv1 == av1) & (bp1 < ap1))
    w1v = jnp.where(swap, bv1, av1)
    w1p = jnp.where(swap, bp1, ap1)
    l1v = jnp.where(swap, av1, bv1)
    l1p = jnp.where(swap, ap1, bp1)
    c2v = jnp.where(swap, bv2, av2)
    c2p = jnp.where(swap, bp2, ap2)
    take2 = (c2v > l1v) | ((c2v == l1v) & (c2p < l1p))
    w2v = jnp.where(take2, c2v, l1v)
    w2p = jnp.where(take2, c2p, l1p)
    return w1v, w1p, w2v, w2p


def _scan_body(step_ref, x_ref,
               bm1_ref, bp1_ref, bm2_ref, bp2_ref, mrow_ref, logs_ref,
               ms_ref, ss_ref):
    vb = pl.program_id(0)
    masking = step_ref[0, 0] < _MINLEN
    sub8 = jax.lax.broadcasted_iota(jnp.int32, (_BEAM, _NR), 0)
    limit = _VOCAB - vb * _VB
    emask = masking & (vb == 0)

    @pl.when(vb == 0)
    def _init():
        ms_ref[...] = jnp.full((2 * _BEAM, _NR), _NEG, jnp.float32)
        ss_ref[...] = jnp.zeros((2 * _BEAM, _NR), jnp.float32)

    ma0 = ms_ref[pl.ds(0, _BEAM), :]
    mb0 = ms_ref[pl.ds(_BEAM, _BEAM), :]
    sa0 = ss_ref[pl.ds(0, _BEAM), :]
    sb0 = ss_ref[pl.ds(_BEAM, _BEAM), :]
    zneg = jnp.full((_BEAM, _NR), _NEG, jnp.float32)
    zi = jnp.zeros((_BEAM, _NR), jnp.int32)

    def one(s, m, ssum, v1, b1, v2, b2):
        x = x_ref[pl.ds(s * _BEAM, _BEAM), :]
        pos = s * _BEAM + sub8
        invalid = pos >= limit
        xs = jnp.where(invalid, _NEG, x)
        xc = jnp.where(invalid | (emask & (pos == _END)), _NEG, x)
        nm = jnp.maximum(m, xs)
        ssum = ssum * jnp.exp(m - nm) + jnp.exp(xs - nm)
        gt1 = xc > v1
        gt2 = xc > v2
        nv1 = jnp.maximum(xc, v1)
        nb1 = jnp.where(gt1, pos, b1)
        nv2 = jnp.where(gt1, v1, jnp.where(gt2, xc, v2))
        nb2 = jnp.where(gt1, b1, jnp.where(gt2, pos, b2))
        return nm, ssum, nv1, nb1, nv2, nb2

    def strip2(i, carry):
        ma, sa, mb, sb, v1, b1, v2, b2 = carry
        ma, sa, v1, b1, v2, b2 = one(2 * i, ma, sa, v1, b1, v2, b2)
        mb, sb, v1, b1, v2, b2 = one(2 * i + 1, mb, sb, v1, b1, v2, b2)
        return ma, sa, mb, sb, v1, b1, v2, b2

    ma, sa, mb, sb, v1, b1, v2, b2 = jax.lax.fori_loop(
        0, _NSTR // 2, strip2, (ma0, sa0, mb0, sb0, zneg, zi, zneg, zi))
    ms_ref[pl.ds(0, _BEAM), :] = ma
    ms_ref[pl.ds(_BEAM, _BEAM), :] = mb
    ss_ref[pl.ds(0, _BEAM), :] = sa
    ss_ref[pl.ds(_BEAM, _BEAM), :] = sb

    h = _BEAM
    while h > 1:
        h //= 2
        v1, b1, v2, b2 = _merge_top2(
            v1[0:h], b1[0:h], v2[0:h], b2[0:h],
            v1[h:2 * h], b1[h:2 * h], v2[h:2 * h], b2[h:2 * h])
    bm1_ref[...] = v1.reshape(1, 1, _NR)
    bp1_ref[...] = b1.reshape(1, 1, _NR)
    bm2_ref[...] = v2.reshape(1, 1, _NR)
    bp2_ref[...] = b2.reshape(1, 1, _NR)

    @pl.when(vb == _NB - 1)
    def _finalize():
        mm = ms_ref[...]
        sF = ss_ref[...]
        mf = mm
        for sh in (8, 4, 2, 1):
            mf = jnp.maximum(mf[0:sh], mf[sh:2 * sh])
        mf16 = jnp.broadcast_to(mf, (2 * _BEAM, _NR))
        corr = sF * jnp.exp(mm - mf16)
        for sh in (8, 4, 2, 1):
            corr = corr[0:sh] + corr[sh:2 * sh]
        mrow_ref[...] = mf.reshape(1, 1, _NR)
        logs_ref[...] = jnp.log(corr).reshape(1, 1, _NR)


def _extract_body(step_ref, pen_ref, c1_ref, q1_ref, c2_ref, q2_ref,
                  mr_ref, ls_ref, tlp_ref, lgt_ref,
                  sc_ref, lp_ref, tok_ref, row_ref, fin_ref,
                  lps_ref, gs_ref, scr_ref, sem):
    masking = step_ref[0, 0] < _MINLEN
    inv_pen = pen_ref[0, 0]
    pid = pl.program_id(0)

    lane98 = jax.lax.broadcasted_iota(jnp.int32, (_BEAM, _NB), 1)
    row8 = jax.lax.broadcasted_iota(jnp.int32, (_BEAM, _NB), 0)
    lane16 = jax.lax.broadcasted_iota(jnp.int32, (2 * _BEAM, _NB), 1)
    row16 = jax.lax.broadcasted_iota(jnp.int32, (2 * _BEAM, _NB), 0)
    row81 = jax.lax.broadcasted_iota(jnp.int32, (_BEAM, 1), 0)

    mrow = mr_ref[...]
    logS = ls_ref[...]
    tlp = tlp_ref[...]
    lp1 = ((c1_ref[...] - mrow) - logS) + tlp
    lp2 = ((c2_ref[...] - mrow) - logS) + tlp
    g1 = row8 * _VOCAB + lane98 * _VB + q1_ref[...]
    g2 = row8 * _VOCAB + lane98 * _VB + q2_ref[...]
    lps_ref[...] = jnp.concatenate([lp1, lp2], axis=0)
    gs_ref[...] = jnp.concatenate([g1, g2], axis=0)

    winners_b = []
    picked_g = []

    for i in range(_BEAM):
        lp_all = lps_ref[...]
        g_all = gs_ref[...]
        w = jnp.max(lp_all)
        gw = jnp.min(jnp.where(lp_all == w, g_all, _BIGI))
        r = gw // _VOCAB
        pos = gw - r * _VOCAB
        vbw = pos // _VB
        bid = r * _NB + vbw
        picked_g.append(gw)

        lps_ref[...] = jnp.where(g_all == gw, jnp.float32(_NEG), lp_all)

        exhausted = jnp.bool_(False)
        for pb in winners_b:
            exhausted = jnp.logical_or(exhausted, pb == bid)
        winners_b.append(bid)

        if i > 0:
            @pl.when(exhausted)
            def _rescan(r=r, vbw=vbw, picked=tuple(picked_g)):
                rg = pid * _BEAM + r
                s0 = jnp.minimum(vbw * _VB, _VOCAB - _VB)
                cp = pltpu.make_async_copy(
                    lgt_ref.at[pl.ds(pl.multiple_of(s0, _BEAM), _VB), :],
                    scr_ref, sem)
                cp.start()
                cp.wait()
                sub8 = jax.lax.broadcasted_iota(jnp.int32, (_BEAM, _NR), 0)
                lanes = jax.lax.broadcasted_iota(jnp.int32, (_BEAM, _NR), 1)

                def stepR(s, carry):
                    vm, pm = carry
                    x = scr_ref[pl.ds(s * _BEAM, _BEAM), :]
                    p = s0 + s * _BEAM + sub8
                    ok = (lanes == rg) & (p // _VB == vbw) & (p < _VOCAB)
                    ok = ok & jnp.logical_not(masking & (p == _END))
                    gv = r * _VOCAB + p
                    for pg in picked:
                        ok = ok & (gv != pg)
                    xv = jnp.where(ok, x, jnp.float32(_NEG))
                    upd = xv > vm
                    return jnp.maximum(xv, vm), jnp.where(upd, p, pm)

                vm = jnp.full((_BEAM, _NR), _NEG, jnp.float32)
                pm = jnp.zeros((_BEAM, _NR), jnp.int32)
                vm, pm = jax.lax.fori_loop(0, _NSTR, stepR, (vm, pm))
                nv = jnp.max(vm)
                np_ = jnp.min(jnp.where(vm == nv, pm, _BIGI))
                m_r = jnp.sum(jnp.where(row81 == r, mrow, 0.0))
                ls_r = jnp.sum(jnp.where(row81 == r, logS, 0.0))
                tl_r = jnp.sum(jnp.where(row81 == r, tlp, 0.0))
                lp_new = ((nv - m_r) - ls_r) + tl_r
                g_new = r * _VOCAB + np_
                sel = (row16 == r) & (lane16 == vbw)
                lps_ref[...] = jnp.where(sel, lp_new, lps_ref[...])
                gs_ref[...] = jnp.where(sel, g_new, gs_ref[...])

        tok = pos
        sc_ref[pl.ds(i, 1), :] = jnp.reshape(w * inv_pen, (1, 1))
        lp_ref[pl.ds(i, 1), :] = jnp.reshape(w, (1, 1))
        tok_ref[pl.ds(i, 1), :] = jnp.reshape(tok, (1, 1))
        row_ref[pl.ds(i, 1), :] = jnp.reshape(r + pid * _BEAM, (1, 1))
        fin_ref[pl.ds(i, 1), :] = jnp.reshape(
            (tok == _END).astype(jnp.int32), (1, 1))


def kernel(logits, topk_log_probs, growing_beam, step):
    nrows = logits.shape[0]
    nb = nrows // _BEAM
    cur_len = growing_beam.shape[1]
    step2d = jnp.reshape(jnp.asarray(step, jnp.int32), (1, 1))
    length_penalty = ((5.0 + (jnp.asarray(step, jnp.int32) + 1)) / 6.0) ** 0.95
    invpen2d = jnp.reshape(
        (1.0 / length_penalty).astype(jnp.float32), (1, 1))
    tlp2d = jnp.reshape(topk_log_probs, (nrows, 1))

    lgt = jnp.swapaxes(logits, 0, 1)

    scan_out = pl.pallas_call(
        _scan_body,
        grid=(_NB,),
        in_specs=[
            pl.BlockSpec(memory_space=pltpu.SMEM),
            pl.BlockSpec((_VB, nrows), lambda vb: (vb, 0)),
        ],
        out_specs=[
            pl.BlockSpec((1, 1, nrows), lambda vb: (vb, 0, 0)),
            pl.BlockSpec((1, 1, nrows), lambda vb: (vb, 0, 0)),
            pl.BlockSpec((1, 1, nrows), lambda vb: (vb, 0, 0)),
            pl.BlockSpec((1, 1, nrows), lambda vb: (vb, 0, 0)),
            pl.BlockSpec((1, 1, nrows), lambda vb: (0, 0, 0)),
            pl.BlockSpec((1, 1, nrows), lambda vb: (0, 0, 0)),
        ],
        scratch_shapes=[
            pltpu.VMEM((2 * _BEAM, nrows), jnp.float32),
            pltpu.VMEM((2 * _BEAM, nrows), jnp.float32),
        ],
        out_shape=[
            jax.ShapeDtypeStruct((_NB, 1, nrows), jnp.float32),
            jax.ShapeDtypeStruct((_NB, 1, nrows), jnp.int32),
            jax.ShapeDtypeStruct((_NB, 1, nrows), jnp.float32),
            jax.ShapeDtypeStruct((_NB, 1, nrows), jnp.int32),
            jax.ShapeDtypeStruct((1, 1, nrows), jnp.float32),
            jax.ShapeDtypeStruct((1, 1, nrows), jnp.float32),
        ],
    )(step2d, lgt)
    bm1, bp1, bm2, bp2, mrow, logs = scan_out

    c1 = bm1.reshape(_NB, nrows).T
    q1 = bp1.reshape(_NB, nrows).T
    c2 = bm2.reshape(_NB, nrows).T
    q2 = bp2.reshape(_NB, nrows).T
    mrT = mrow.reshape(nrows, 1)
    lsT = logs.reshape(nrows, 1)

    out_shape = [
        jax.ShapeDtypeStruct((nrows, 1), jnp.float32),
        jax.ShapeDtypeStruct((nrows, 1), jnp.float32),
        jax.ShapeDtypeStruct((nrows, 1), jnp.int32),
        jax.ShapeDtypeStruct((nrows, 1), jnp.int32),
        jax.ShapeDtypeStruct((nrows, 1), jnp.int32),
    ]
    sc, lp, tok, rows, fin = pl.pallas_call(
        _extract_body,
        grid=(nb,),
        in_specs=[
            pl.BlockSpec(memory_space=pltpu.SMEM),
            pl.BlockSpec(memory_space=pltpu.SMEM),
            pl.BlockSpec((_BEAM, _NB), lambda b: (b, 0)),
            pl.BlockSpec((_BEAM, _NB), lambda b: (b, 0)),
            pl.BlockSpec((_BEAM, _NB), lambda b: (b, 0)),
            pl.BlockSpec((_BEAM, _NB), lambda b: (b, 0)),
            pl.BlockSpec((_BEAM, 1), lambda b: (b, 0)),
            pl.BlockSpec((_BEAM, 1), lambda b: (b, 0)),
            pl.BlockSpec((_BEAM, 1), lambda b: (b, 0)),
            pl.BlockSpec(memory_space=pl.ANY),
        ],
        out_specs=[
            pl.BlockSpec((_BEAM, 1), lambda b: (b, 0)),
            pl.BlockSpec((_BEAM, 1), lambda b: (b, 0)),
            pl.BlockSpec((_BEAM, 1), lambda b: (b, 0)),
            pl.BlockSpec((_BEAM, 1), lambda b: (b, 0)),
            pl.BlockSpec((_BEAM, 1), lambda b: (b, 0)),
        ],
        scratch_shapes=[
            pltpu.VMEM((2 * _BEAM, _NB), jnp.float32),
            pltpu.VMEM((2 * _BEAM, _NB), jnp.int32),
            pltpu.VMEM((_VB, nrows), jnp.float32),
            pltpu.SemaphoreType.DMA,
        ],
        out_shape=out_shape,
    )(step2d, invpen2d, c1, q1, c2, q2, mrT, lsT, tlp2d, lgt)

    rows_flat = rows.reshape(-1)
    gb_pad = jnp.pad(growing_beam, ((0, 0), (0, _GW - cur_len)))
    hist = _make_sc_gather(nrows)(gb_pad, rows_flat)
    nbm = jnp.concatenate([hist[:, :cur_len], tok], axis=1)

    return (sc.reshape(nb, _BEAM), lp.reshape(nb, _BEAM),
            tok.reshape(nb, _BEAM), rows_flat, nbm,
            (fin.reshape(nb, _BEAM) != 0))


# speedup vs baseline: 1.7515x; 1.7515x over previous
import functools

import jax
import jax.numpy as jnp
from jax import lax
from jax.experimental import pallas as pl
from jax.experimental.pallas import tpu as pltpu
from jax.experimental.pallas import tpu_sc as plsc

_BEAM = 8
_VOCAB = 100000
_NR = 512
_VB = 1024
_NB = -(-_VOCAB // _VB)
_NSTR = _VB // _BEAM
_END = 2
_MINLEN = 10
_NEG = -1e30
_BIGI = 1 << 30
_GW = 128


@functools.cache
def _make_sc_gather(nrows):
    info = plsc.get_sparse_core_info()
    nw = info.num_cores * info.num_subcores
    bpw = nrows // nw
    mesh = plsc.VectorSubcoreMesh(core_axis_name="c", subcore_axis_name="s")

    @functools.partial(
        pl.kernel,
        out_type=jax.ShapeDtypeStruct((nrows, _GW), jnp.int32),
        mesh=mesh,
        scratch_types=[
            pltpu.VMEM((bpw,), jnp.int32),
            pltpu.VMEM((bpw, _GW), jnp.int32),
            pltpu.SemaphoreType.DMA,
        ],
    )
    def sc_gather(gb_hbm, idx_hbm, out_hbm, idx_v, rows_v, sem):
        wid = lax.axis_index("s") * info.num_cores + lax.axis_index("c")
        base = wid * bpw
        pltpu.sync_copy(idx_hbm.at[pl.ds(base, bpw)], idx_v)
        pltpu.async_copy(gb_hbm.at[idx_v], rows_v, sem).wait()
        pltpu.sync_copy(rows_v, out_hbm.at[pl.ds(base, bpw)])

    return sc_gather


def _merge_top2(av1, ap1, av2, ap2, bv1, bp1, bv2, bp2):
    swap = (bv1 > av1) | ((bv1 == av1) & (bp1 < ap1))
    w1v = jnp.where(swap, bv1, av1)
    w1p = jnp.where(swap, bp1, ap1)
    l1v = jnp.where(swap, av1, bv1)
    l1p = jnp.where(swap, ap1, bp1)
    c2v = jnp.where(swap, bv2, av2)
    c2p = jnp.where(swap, bp2, ap2)
    take2 = (c2v > l1v) | ((c2v == l1v) & (c2p < l1p))
    w2v = jnp.where(take2, c2v, l1v)
    w2p = jnp.where(take2, c2p, l1p)
    return w1v, w1p, w2v, w2p


def _scan_body(step_ref, x_ref,
               bm1_ref, bp1_ref, bm2_ref, bp2_ref, mrow_ref, logs_ref,
               ms_ref, ss_ref):
    vb = pl.program_id(0)
    masking = step_ref[0, 0] < _MINLEN
    sub8 = jax.lax.broadcasted_iota(jnp.int32, (_BEAM, _NR), 0)
    limit = _VOCAB - vb * _VB
    emask = masking & (vb == 0)

    @pl.when(vb == 0)
    def _init():
        ms_ref[...] = jnp.full((2 * _BEAM, _NR), _NEG, jnp.float32)
        ss_ref[...] = jnp.zeros((2 * _BEAM, _NR), jnp.float32)

    ma0 = ms_ref[pl.ds(0, _BEAM), :]
    mb0 = ms_ref[pl.ds(_BEAM, _BEAM), :]
    sa0 = ss_ref[pl.ds(0, _BEAM), :]
    sb0 = ss_ref[pl.ds(_BEAM, _BEAM), :]
    zneg = jnp.full((_BEAM, _NR), _NEG, jnp.float32)
    zi = jnp.zeros((_BEAM, _NR), jnp.int32)

    def one(s, m, ssum, v1, b1, v2, b2):
        x = x_ref[pl.ds(s * _BEAM, _BEAM), :]
        pos = s * _BEAM + sub8
        invalid = pos >= limit
        xs = jnp.where(invalid, _NEG, x)
        xc = jnp.where(invalid | (emask & (pos == _END)), _NEG, x)
        nm = jnp.maximum(m, xs)
        ssum = ssum * jnp.exp(m - nm) + jnp.exp(xs - nm)
        gt1 = xc > v1
        gt2 = xc > v2
        nv1 = jnp.maximum(xc, v1)
        nb1 = jnp.where(gt1, pos, b1)
        nv2 = jnp.where(gt1, v1, jnp.where(gt2, xc, v2))
        nb2 = jnp.where(gt1, b1, jnp.where(gt2, pos, b2))
        return nm, ssum, nv1, nb1, nv2, nb2

    def strip2(i, carry):
        ma, sa, mb, sb, v1, b1, v2, b2 = carry
        ma, sa, v1, b1, v2, b2 = one(2 * i, ma, sa, v1, b1, v2, b2)
        mb, sb, v1, b1, v2, b2 = one(2 * i + 1, mb, sb, v1, b1, v2, b2)
        return ma, sa, mb, sb, v1, b1, v2, b2

    ma, sa, mb, sb, v1, b1, v2, b2 = jax.lax.fori_loop(
        0, _NSTR // 2, strip2, (ma0, sa0, mb0, sb0, zneg, zi, zneg, zi))
    ms_ref[pl.ds(0, _BEAM), :] = ma
    ms_ref[pl.ds(_BEAM, _BEAM), :] = mb
    ss_ref[pl.ds(0, _BEAM), :] = sa
    ss_ref[pl.ds(_BEAM, _BEAM), :] = sb

    h = _BEAM
    while h > 1:
        h //= 2
        v1, b1, v2, b2 = _merge_top2(
            v1[0:h], b1[0:h], v2[0:h], b2[0:h],
            v1[h:2 * h], b1[h:2 * h], v2[h:2 * h], b2[h:2 * h])
    bm1_ref[...] = v1.reshape(1, 1, _NR)
    bp1_ref[...] = b1.reshape(1, 1, _NR)
    bm2_ref[...] = v2.reshape(1, 1, _NR)
    bp2_ref[...] = b2.reshape(1, 1, _NR)

    @pl.when(vb == _NB - 1)
    def _finalize():
        mm = ms_ref[...]
        sF = ss_ref[...]
        mf = mm
        for sh in (8, 4, 2, 1):
            mf = jnp.maximum(mf[0:sh], mf[sh:2 * sh])
        mf16 = jnp.broadcast_to(mf, (2 * _BEAM, _NR))
        corr = sF * jnp.exp(mm - mf16)
        for sh in (8, 4, 2, 1):
            corr = corr[0:sh] + corr[sh:2 * sh]
        mrow_ref[...] = mf.reshape(1, 1, _NR)
        logs_ref[...] = jnp.log(corr).reshape(1, 1, _NR)


def _extract_body(step_ref, pen_ref, c1_ref, q1_ref, c2_ref, q2_ref,
                  mr_ref, ls_ref, tlp_ref, lgt_ref,
                  sc_ref, lp_ref, tok_ref, row_ref, fin_ref,
                  lps_ref, gs_ref, scr_ref, sem):
    masking = step_ref[0, 0] < _MINLEN
    inv_pen = pen_ref[0, 0]
    pid = pl.program_id(0)

    lane98 = jax.lax.broadcasted_iota(jnp.int32, (_BEAM, _NB), 1)
    row8 = jax.lax.broadcasted_iota(jnp.int32, (_BEAM, _NB), 0)
    lane16 = jax.lax.broadcasted_iota(jnp.int32, (2 * _BEAM, _NB), 1)
    row16 = jax.lax.broadcasted_iota(jnp.int32, (2 * _BEAM, _NB), 0)
    row81 = jax.lax.broadcasted_iota(jnp.int32, (_BEAM, 1), 0)

    mrow = mr_ref[...]
    logS = ls_ref[...]
    tlp = tlp_ref[...]
    lp1 = ((c1_ref[...] - mrow) - logS) + tlp
    lp2 = ((c2_ref[...] - mrow) - logS) + tlp
    g1 = row8 * _VOCAB + lane98 * _VB + q1_ref[...]
    g2 = row8 * _VOCAB + lane98 * _VB + q2_ref[...]
    lps_ref[...] = jnp.concatenate([lp1, lp2], axis=0)
    gs_ref[...] = jnp.concatenate([g1, g2], axis=0)

    winners_b = []
    picked_g = []
    wcol = jnp.zeros((_BEAM, 1), jnp.float32)
    gcol = jnp.zeros((_BEAM, 1), jnp.int32)

    for i in range(_BEAM):
        lp_all = lps_ref[...]
        g_all = gs_ref[...]
        w = jnp.max(lp_all)
        gw = jnp.min(jnp.where(lp_all == w, g_all, _BIGI))
        lps_ref[...] = jnp.where(g_all == gw, jnp.float32(_NEG), lp_all)

        wcol = wcol + jnp.where(row81 == i, w, 0.0)
        gcol = gcol + jnp.where(row81 == i, gw, 0)

        r = gw // _VOCAB
        pos = gw - r * _VOCAB
        vbw = pos // _VB
        bid = r * _NB + vbw
        picked_g.append(gw)

        exhausted = jnp.bool_(False)
        for pb in winners_b:
            exhausted = jnp.logical_or(exhausted, pb == bid)
        winners_b.append(bid)

        if i > 0:
            @pl.when(exhausted)
            def _rescan(r=r, vbw=vbw, picked=tuple(picked_g)):
                rg = pid * _BEAM + r
                s0 = jnp.minimum(vbw * _VB, _VOCAB - _VB)
                cp = pltpu.make_async_copy(
                    lgt_ref.at[pl.ds(pl.multiple_of(s0, _BEAM), _VB), :],
                    scr_ref, sem)
                cp.start()
                cp.wait()
                sub8 = jax.lax.broadcasted_iota(jnp.int32, (_BEAM, _NR), 0)
                lanes = jax.lax.broadcasted_iota(jnp.int32, (_BEAM, _NR), 1)

                def stepR(s, carry):
                    vm, pm = carry
                    x = scr_ref[pl.ds(s * _BEAM, _BEAM), :]
                    p = s0 + s * _BEAM + sub8
                    ok = (lanes == rg) & (p // _VB == vbw) & (p < _VOCAB)
                    ok = ok & jnp.logical_not(masking & (p == _END))
                    gv = r * _VOCAB + p
                    for pg in picked:
                        ok = ok & (gv != pg)
                    xv = jnp.where(ok, x, jnp.float32(_NEG))
                    upd = xv > vm
                    return jnp.maximum(xv, vm), jnp.where(upd, p, pm)

                vm = jnp.full((_BEAM, _NR), _NEG, jnp.float32)
                pm = jnp.zeros((_BEAM, _NR), jnp.int32)
                vm, pm = jax.lax.fori_loop(0, _NSTR, stepR, (vm, pm))
                nv = jnp.max(vm)
                np_ = jnp.min(jnp.where(vm == nv, pm, _BIGI))
                m_r = jnp.sum(jnp.where(row81 == r, mrow, 0.0))
                ls_r = jnp.sum(jnp.where(row81 == r, logS, 0.0))
                tl_r = jnp.sum(jnp.where(row81 == r, tlp, 0.0))
                lp_new = ((nv - m_r) - ls_r) + tl_r
                g_new = r * _VOCAB + np_
                sel = (row16 == r) & (lane16 == vbw)
                lps_ref[...] = jnp.where(sel, lp_new, lps_ref[...])
                gs_ref[...] = jnp.where(sel, g_new, gs_ref[...])

    rcol = gcol // _VOCAB
    tokcol = gcol - rcol * _VOCAB
    sc_ref[...] = wcol * inv_pen
    lp_ref[...] = wcol
    tok_ref[...] = tokcol
    row_ref[...] = rcol + pid * _BEAM
    fin_ref[...] = (tokcol == _END).astype(jnp.int32)


def kernel(logits, topk_log_probs, growing_beam, step):
    nrows = logits.shape[0]
    nb = nrows // _BEAM
    cur_len = growing_beam.shape[1]
    step2d = jnp.reshape(jnp.asarray(step, jnp.int32), (1, 1))
    length_penalty = ((5.0 + (jnp.asarray(step, jnp.int32) + 1)) / 6.0) ** 0.95
    invpen2d = jnp.reshape(
        (1.0 / length_penalty).astype(jnp.float32), (1, 1))
    tlp2d = jnp.reshape(topk_log_probs, (nrows, 1))

    lgt = jnp.swapaxes(logits, 0, 1)

    scan_out = pl.pallas_call(
        _scan_body,
        grid=(_NB,),
        in_specs=[
            pl.BlockSpec(memory_space=pltpu.SMEM),
            pl.BlockSpec((_VB, nrows), lambda vb: (vb, 0)),
        ],
        out_specs=[
            pl.BlockSpec((1, 1, nrows), lambda vb: (vb, 0, 0)),
            pl.BlockSpec((1, 1, nrows), lambda vb: (vb, 0, 0)),
            pl.BlockSpec((1, 1, nrows), lambda vb: (vb, 0, 0)),
            pl.BlockSpec((1, 1, nrows), lambda vb: (vb, 0, 0)),
            pl.BlockSpec((1, 1, nrows), lambda vb: (0, 0, 0)),
            pl.BlockSpec((1, 1, nrows), lambda vb: (0, 0, 0)),
        ],
        scratch_shapes=[
            pltpu.VMEM((2 * _BEAM, nrows), jnp.float32),
            pltpu.VMEM((2 * _BEAM, nrows), jnp.float32),
        ],
        out_shape=[
            jax.ShapeDtypeStruct((_NB, 1, nrows), jnp.float32),
            jax.ShapeDtypeStruct((_NB, 1, nrows), jnp.int32),
            jax.ShapeDtypeStruct((_NB, 1, nrows), jnp.float32),
            jax.ShapeDtypeStruct((_NB, 1, nrows), jnp.int32),
            jax.ShapeDtypeStruct((1, 1, nrows), jnp.float32),
            jax.ShapeDtypeStruct((1, 1, nrows), jnp.float32),
        ],
    )(step2d, lgt)
    bm1, bp1, bm2, bp2, mrow, logs = scan_out

    c1 = bm1.reshape(_NB, nrows).T
    q1 = bp1.reshape(_NB, nrows).T
    c2 = bm2.reshape(_NB, nrows).T
    q2 = bp2.reshape(_NB, nrows).T
    mrT = mrow.reshape(nrows, 1)
    lsT = logs.reshape(nrows, 1)

    out_shape = [
        jax.ShapeDtypeStruct((nrows, 1), jnp.float32),
        jax.ShapeDtypeStruct((nrows, 1), jnp.float32),
        jax.ShapeDtypeStruct((nrows, 1), jnp.int32),
        jax.ShapeDtypeStruct((nrows, 1), jnp.int32),
        jax.ShapeDtypeStruct((nrows, 1), jnp.int32),
    ]
    sc, lp, tok, rows, fin = pl.pallas_call(
        _extract_body,
        grid=(nb,),
        in_specs=[
            pl.BlockSpec(memory_space=pltpu.SMEM),
            pl.BlockSpec(memory_space=pltpu.SMEM),
            pl.BlockSpec((_BEAM, _NB), lambda b: (b, 0)),
            pl.BlockSpec((_BEAM, _NB), lambda b: (b, 0)),
            pl.BlockSpec((_BEAM, _NB), lambda b: (b, 0)),
            pl.BlockSpec((_BEAM, _NB), lambda b: (b, 0)),
            pl.BlockSpec((_BEAM, 1), lambda b: (b, 0)),
            pl.BlockSpec((_BEAM, 1), lambda b: (b, 0)),
            pl.BlockSpec((_BEAM, 1), lambda b: (b, 0)),
            pl.BlockSpec(memory_space=pl.ANY),
        ],
        out_specs=[
            pl.BlockSpec((_BEAM, 1), lambda b: (b, 0)),
            pl.BlockSpec((_BEAM, 1), lambda b: (b, 0)),
            pl.BlockSpec((_BEAM, 1), lambda b: (b, 0)),
            pl.BlockSpec((_BEAM, 1), lambda b: (b, 0)),
            pl.BlockSpec((_BEAM, 1), lambda b: (b, 0)),
        ],
        scratch_shapes=[
            pltpu.VMEM((2 * _BEAM, _NB), jnp.float32),
            pltpu.VMEM((2 * _BEAM, _NB), jnp.int32),
            pltpu.VMEM((_VB, nrows), jnp.float32),
            pltpu.SemaphoreType.DMA,
        ],
        out_shape=out_shape,
    )(step2d, invpen2d, c1, q1, c2, q2, mrT, lsT, tlp2d, lgt)

    rows_flat = rows.reshape(-1)
    gb_pad = jnp.pad(growing_beam, ((0, 0), (0, _GW - cur_len)))
    hist = _make_sc_gather(nrows)(gb_pad, rows_flat)
    nbm = jnp.concatenate([hist[:, :cur_len], tok], axis=1)

    return (sc.reshape(nb, _BEAM), lp.reshape(nb, _BEAM),
            tok.reshape(nb, _BEAM), rows_flat, nbm,
            (fin.reshape(nb, _BEAM) != 0))


# speedup vs baseline: 1.7577x; 1.0036x over previous
import functools

import jax
import jax.numpy as jnp
from jax import lax
from jax.experimental import pallas as pl
from jax.experimental.pallas import tpu as pltpu
from jax.experimental.pallas import tpu_sc as plsc

_BEAM = 8
_VOCAB = 100000
_NR = 512
_VB = 1024
_NB = -(-_VOCAB // _VB)
_NSTR = _VB // _BEAM
_END = 2
_MINLEN = 10
_NEG = -1e30
_BIGI = 1 << 30
_GW = 128


@functools.cache
def _make_sc_gather(nrows):
    info = plsc.get_sparse_core_info()
    nw = info.num_cores * info.num_subcores
    bpw = nrows // nw
    mesh = plsc.VectorSubcoreMesh(core_axis_name="c", subcore_axis_name="s")

    @functools.partial(
        pl.kernel,
        out_type=jax.ShapeDtypeStruct((nrows, _GW), jnp.int32),
        mesh=mesh,
        scratch_types=[
            pltpu.VMEM((bpw,), jnp.int32),
            pltpu.VMEM((bpw, _GW), jnp.int32),
            pltpu.SemaphoreType.DMA,
        ],
    )
    def sc_gather(gb_hbm, idx_hbm, out_hbm, idx_v, rows_v, sem):
        wid = lax.axis_index("s") * info.num_cores + lax.axis_index("c")
        base = wid * bpw
        pltpu.sync_copy(idx_hbm.at[pl.ds(base, bpw)], idx_v)
        pltpu.async_copy(gb_hbm.at[idx_v], rows_v, sem).wait()
        pltpu.sync_copy(rows_v, out_hbm.at[pl.ds(base, bpw)])

    return sc_gather


def _merge_top2(av1, ap1, av2, ap2, bv1, bp1, bv2, bp2):
    swap = (bv1 > av1) | ((bv1 == av1) & (bp1 < ap1))
    w1v = jnp.where(swap, bv1, av1)
    w1p = jnp.where(swap, bp1, ap1)
    l1v = jnp.where(swap, av1, bv1)
    l1p = jnp.where(swap, ap1, bp1)
    c2v = jnp.where(swap, bv2, av2)
    c2p = jnp.where(swap, bp2, ap2)
    take2 = (c2v > l1v) | ((c2v == l1v) & (c2p < l1p))
    w2v = jnp.where(take2, c2v, l1v)
    w2p = jnp.where(take2, c2p, l1p)
    return w1v, w1p, w2v, w2p


def _scan_body(step_ref, x_ref,
               bm1_ref, bp1_ref, bm2_ref, bp2_ref, mrow_ref, logs_ref,
               ms_ref, ss_ref):
    vb = pl.program_id(0)
    masking = step_ref[0, 0] < _MINLEN
    sub8 = jax.lax.broadcasted_iota(jnp.int32, (_BEAM, _NR), 0)
    limit = _VOCAB - vb * _VB
    emask = masking & (vb == 0)

    @pl.when(vb == 0)
    def _init():
        ms_ref[...] = jnp.full((2 * _BEAM, _NR), _NEG, jnp.float32)
        ss_ref[...] = jnp.zeros((2 * _BEAM, _NR), jnp.float32)

    ma0 = ms_ref[pl.ds(0, _BEAM), :]
    mb0 = ms_ref[pl.ds(_BEAM, _BEAM), :]
    sa0 = ss_ref[pl.ds(0, _BEAM), :]
    sb0 = ss_ref[pl.ds(_BEAM, _BEAM), :]
    zneg = jnp.full((_BEAM, _NR), _NEG, jnp.float32)
    zi = jnp.zeros((_BEAM, _NR), jnp.int32)

    def one(s, m, ssum, v1, b1, v2, b2):
        x = x_ref[pl.ds(s * _BEAM, _BEAM), :]
        pos = s * _BEAM + sub8
        invalid = pos >= limit
        xs = jnp.where(invalid, _NEG, x)
        xc = jnp.where(invalid | (emask & (pos == _END)), _NEG, x)
        nm = jnp.maximum(m, xs)
        ssum = ssum * jnp.exp(m - nm) + jnp.exp(xs - nm)
        gt1 = xc > v1
        gt2 = xc > v2
        nv1 = jnp.maximum(xc, v1)
        nb1 = jnp.where(gt1, pos, b1)
        nv2 = jnp.where(gt1, v1, jnp.where(gt2, xc, v2))
        nb2 = jnp.where(gt1, b1, jnp.where(gt2, pos, b2))
        return nm, ssum, nv1, nb1, nv2, nb2

    def strip2(i, carry):
        ma, sa, mb, sb, v1, b1, v2, b2 = carry
        ma, sa, v1, b1, v2, b2 = one(2 * i, ma, sa, v1, b1, v2, b2)
        mb, sb, v1, b1, v2, b2 = one(2 * i + 1, mb, sb, v1, b1, v2, b2)
        return ma, sa, mb, sb, v1, b1, v2, b2

    ma, sa, mb, sb, v1, b1, v2, b2 = jax.lax.fori_loop(
        0, _NSTR // 2, strip2, (ma0, sa0, mb0, sb0, zneg, zi, zneg, zi))
    ms_ref[pl.ds(0, _BEAM), :] = ma
    ms_ref[pl.ds(_BEAM, _BEAM), :] = mb
    ss_ref[pl.ds(0, _BEAM), :] = sa
    ss_ref[pl.ds(_BEAM, _BEAM), :] = sb

    h = _BEAM
    while h > 1:
        h //= 2
        v1, b1, v2, b2 = _merge_top2(
            v1[0:h], b1[0:h], v2[0:h], b2[0:h],
            v1[h:2 * h], b1[h:2 * h], v2[h:2 * h], b2[h:2 * h])
    bm1_ref[...] = v1.reshape(1, 1, _NR)
    bp1_ref[...] = b1.reshape(1, 1, _NR)
    bm2_ref[...] = v2.reshape(1, 1, _NR)
    bp2_ref[...] = b2.reshape(1, 1, _NR)

    @pl.when(vb == _NB - 1)
    def _finalize():
        mm = ms_ref[...]
        sF = ss_ref[...]
        mf = mm
        for sh in (8, 4, 2, 1):
            mf = jnp.maximum(mf[0:sh], mf[sh:2 * sh])
        mf16 = jnp.broadcast_to(mf, (2 * _BEAM, _NR))
        corr = sF * jnp.exp(mm - mf16)
        for sh in (8, 4, 2, 1):
            corr = corr[0:sh] + corr[sh:2 * sh]
        mrow_ref[...] = mf.reshape(1, 1, _NR)
        logs_ref[...] = jnp.log(corr).reshape(1, 1, _NR)


def _extract_body(step_ref, pen_ref, c1_ref, q1_ref, c2_ref, q2_ref,
                  mr_ref, ls_ref, tlp_ref, lgt_ref,
                  sc_ref, lp_ref, tok_ref, row_ref, fin_ref,
                  lps_ref, gs_ref, scr_ref, sem):
    masking = step_ref[0, 0] < _MINLEN
    inv_pen = pen_ref[0, 0]

    lane98 = jax.lax.broadcasted_iota(jnp.int32, (_BEAM, _NB), 1)
    row8 = jax.lax.broadcasted_iota(jnp.int32, (_BEAM, _NB), 0)
    lane16 = jax.lax.broadcasted_iota(jnp.int32, (2 * _BEAM, _NB), 1)
    row16 = jax.lax.broadcasted_iota(jnp.int32, (2 * _BEAM, _NB), 0)
    row81 = jax.lax.broadcasted_iota(jnp.int32, (_BEAM, 1), 0)

    def batch(b, carry):
        base = pl.multiple_of(b * _BEAM, _BEAM)
        mrow = mr_ref[pl.ds(base, _BEAM), :]
        logS = ls_ref[pl.ds(base, _BEAM), :]
        tlp = tlp_ref[pl.ds(base, _BEAM), :]
        lp1 = ((c1_ref[pl.ds(base, _BEAM), :] - mrow) - logS) + tlp
        lp2 = ((c2_ref[pl.ds(base, _BEAM), :] - mrow) - logS) + tlp
        g1 = row8 * _VOCAB + lane98 * _VB + q1_ref[pl.ds(base, _BEAM), :]
        g2 = row8 * _VOCAB + lane98 * _VB + q2_ref[pl.ds(base, _BEAM), :]
        lps_ref[...] = jnp.concatenate([lp1, lp2], axis=0)
        gs_ref[...] = jnp.concatenate([g1, g2], axis=0)

        winners_b = []
        picked_g = []
        wcol = jnp.zeros((_BEAM, 1), jnp.float32)
        gcol = jnp.zeros((_BEAM, 1), jnp.int32)

        for i in range(_BEAM):
            lp_all = lps_ref[...]
            g_all = gs_ref[...]
            w = jnp.max(lp_all)
            gw = jnp.min(jnp.where(lp_all == w, g_all, _BIGI))
            lps_ref[...] = jnp.where(g_all == gw, jnp.float32(_NEG), lp_all)

            wcol = wcol + jnp.where(row81 == i, w, 0.0)
            gcol = gcol + jnp.where(row81 == i, gw, 0)

            r = gw // _VOCAB
            pos = gw - r * _VOCAB
            vbw = pos // _VB
            bid = r * _NB + vbw
            picked_g.append(gw)

            exhausted = jnp.bool_(False)
            for pb in winners_b:
                exhausted = jnp.logical_or(exhausted, pb == bid)
            winners_b.append(bid)

            if i > 0:
                @pl.when(exhausted)
                def _rescan(r=r, vbw=vbw, picked=tuple(picked_g),
                            mrow=mrow, logS=logS, tlp=tlp):
                    rg = b * _BEAM + r
                    s0 = jnp.minimum(vbw * _VB, _VOCAB - _VB)
                    cp = pltpu.make_async_copy(
                        lgt_ref.at[pl.ds(pl.multiple_of(s0, _BEAM), _VB), :],
                        scr_ref, sem)
                    cp.start()
                    cp.wait()
                    sub8 = jax.lax.broadcasted_iota(jnp.int32, (_BEAM, _NR), 0)
                    lanes = jax.lax.broadcasted_iota(jnp.int32, (_BEAM, _NR), 1)

                    def stepR(s, carry):
                        vm, pm = carry
                        x = scr_ref[pl.ds(s * _BEAM, _BEAM), :]
                        p = s0 + s * _BEAM + sub8
                        ok = (lanes == rg) & (p // _VB == vbw) & (p < _VOCAB)
                        ok = ok & jnp.logical_not(masking & (p == _END))
                        gv = r * _VOCAB + p
                        for pg in picked:
                            ok = ok & (gv != pg)
                        xv = jnp.where(ok, x, jnp.float32(_NEG))
                        upd = xv > vm
                        return jnp.maximum(xv, vm), jnp.where(upd, p, pm)

                    vm = jnp.full((_BEAM, _NR), _NEG, jnp.float32)
                    pm = jnp.zeros((_BEAM, _NR), jnp.int32)
                    vm, pm = jax.lax.fori_loop(0, _NSTR, stepR, (vm, pm))
                    nv = jnp.max(vm)
                    np_ = jnp.min(jnp.where(vm == nv, pm, _BIGI))
                    m_r = jnp.sum(jnp.where(row81 == r, mrow, 0.0))
                    ls_r = jnp.sum(jnp.where(row81 == r, logS, 0.0))
                    tl_r = jnp.sum(jnp.where(row81 == r, tlp, 0.0))
                    lp_new = ((nv - m_r) - ls_r) + tl_r
                    g_new = r * _VOCAB + np_
                    sel = (row16 == r) & (lane16 == vbw)
                    lps_ref[...] = jnp.where(sel, lp_new, lps_ref[...])
                    gs_ref[...] = jnp.where(sel, g_new, gs_ref[...])

        rcol = gcol // _VOCAB
        tokcol = gcol - rcol * _VOCAB
        sc_ref[pl.ds(base, _BEAM), :] = wcol * inv_pen
        lp_ref[pl.ds(base, _BEAM), :] = wcol
        tok_ref[pl.ds(base, _BEAM), :] = tokcol
        row_ref[pl.ds(base, _BEAM), :] = rcol + b * _BEAM
        fin_ref[pl.ds(base, _BEAM), :] = (tokcol == _END).astype(jnp.int32)
        return carry

    jax.lax.fori_loop(0, _NR // _BEAM, batch, 0)


def kernel(logits, topk_log_probs, growing_beam, step):
    nrows = logits.shape[0]
    nb = nrows // _BEAM
    cur_len = growing_beam.shape[1]
    step2d = jnp.reshape(jnp.asarray(step, jnp.int32), (1, 1))
    length_penalty = ((5.0 + (jnp.asarray(step, jnp.int32) + 1)) / 6.0) ** 0.95
    invpen2d = jnp.reshape(
        (1.0 / length_penalty).astype(jnp.float32), (1, 1))
    tlp2d = jnp.reshape(topk_log_probs, (nrows, 1))

    lgt = jnp.swapaxes(logits, 0, 1)

    scan_out = pl.pallas_call(
        _scan_body,
        grid=(_NB,),
        in_specs=[
            pl.BlockSpec(memory_space=pltpu.SMEM),
            pl.BlockSpec((_VB, nrows), lambda vb: (vb, 0)),
        ],
        out_specs=[
            pl.BlockSpec((1, 1, nrows), lambda vb: (vb, 0, 0)),
            pl.BlockSpec((1, 1, nrows), lambda vb: (vb, 0, 0)),
            pl.BlockSpec((1, 1, nrows), lambda vb: (vb, 0, 0)),
            pl.BlockSpec((1, 1, nrows), lambda vb: (vb, 0, 0)),
            pl.BlockSpec((1, 1, nrows), lambda vb: (0, 0, 0)),
            pl.BlockSpec((1, 1, nrows), lambda vb: (0, 0, 0)),
        ],
        scratch_shapes=[
            pltpu.VMEM((2 * _BEAM, nrows), jnp.float32),
            pltpu.VMEM((2 * _BEAM, nrows), jnp.float32),
        ],
        out_shape=[
            jax.ShapeDtypeStruct((_NB, 1, nrows), jnp.float32),
            jax.ShapeDtypeStruct((_NB, 1, nrows), jnp.int32),
            jax.ShapeDtypeStruct((_NB, 1, nrows), jnp.float32),
            jax.ShapeDtypeStruct((_NB, 1, nrows), jnp.int32),
            jax.ShapeDtypeStruct((1, 1, nrows), jnp.float32),
            jax.ShapeDtypeStruct((1, 1, nrows), jnp.float32),
        ],
    )(step2d, lgt)
    bm1, bp1, bm2, bp2, mrow, logs = scan_out

    c1 = bm1.reshape(_NB, nrows).T
    q1 = bp1.reshape(_NB, nrows).T
    c2 = bm2.reshape(_NB, nrows).T
    q2 = bp2.reshape(_NB, nrows).T
    mrT = mrow.reshape(nrows, 1)
    lsT = logs.reshape(nrows, 1)

    out_shape = [
        jax.ShapeDtypeStruct((nrows, 1), jnp.float32),
        jax.ShapeDtypeStruct((nrows, 1), jnp.float32),
        jax.ShapeDtypeStruct((nrows, 1), jnp.int32),
        jax.ShapeDtypeStruct((nrows, 1), jnp.int32),
        jax.ShapeDtypeStruct((nrows, 1), jnp.int32),
    ]
    sc, lp, tok, rows, fin = pl.pallas_call(
        _extract_body,
        in_specs=[
            pl.BlockSpec(memory_space=pltpu.SMEM),
            pl.BlockSpec(memory_space=pltpu.SMEM),
            pl.BlockSpec(),
            pl.BlockSpec(),
            pl.BlockSpec(),
            pl.BlockSpec(),
            pl.BlockSpec(),
            pl.BlockSpec(),
            pl.BlockSpec(),
            pl.BlockSpec(memory_space=pl.ANY),
        ],
        out_specs=[
            pl.BlockSpec(),
            pl.BlockSpec(),
            pl.BlockSpec(),
            pl.BlockSpec(),
            pl.BlockSpec(),
        ],
        scratch_shapes=[
            pltpu.VMEM((2 * _BEAM, _NB), jnp.float32),
            pltpu.VMEM((2 * _BEAM, _NB), jnp.int32),
            pltpu.VMEM((_VB, nrows), jnp.float32),
            pltpu.SemaphoreType.DMA,
        ],
        out_shape=out_shape,
    )(step2d, invpen2d, c1, q1, c2, q2, mrT, lsT, tlp2d, lgt)

    rows_flat = rows.reshape(-1)
    gb_pad = jnp.pad(growing_beam, ((0, 0), (0, _GW - cur_len)))
    hist = _make_sc_gather(nrows)(gb_pad, rows_flat)
    nbm = jnp.concatenate([hist[:, :cur_len], tok], axis=1)

    return (sc.reshape(nb, _BEAM), lp.reshape(nb, _BEAM),
            tok.reshape(nb, _BEAM), rows_flat, nbm,
            (fin.reshape(nb, _BEAM) != 0))


# speedup vs baseline: 1.9347x; 1.1007x over previous
import functools

import jax
import jax.numpy as jnp
from jax import lax
from jax.experimental import pallas as pl
from jax.experimental.pallas import tpu as pltpu
from jax.experimental.pallas import tpu_sc as plsc

_BEAM = 8
_VOCAB = 100000
_NR = 512
_VB = 1024
_NB = -(-_VOCAB // _VB)
_NSTR = _VB // _BEAM
_END = 2
_MINLEN = 10
_NEG = -1e30
_BIGI = 1 << 30
_GW = 128


@functools.cache
def _make_sc_gather(nrows):
    info = plsc.get_sparse_core_info()
    nw = info.num_cores * info.num_subcores
    bpw = nrows // nw
    mesh = plsc.VectorSubcoreMesh(core_axis_name="c", subcore_axis_name="s")

    @functools.partial(
        pl.kernel,
        out_type=jax.ShapeDtypeStruct((nrows, _GW), jnp.int32),
        mesh=mesh,
        scratch_types=[
            pltpu.VMEM((bpw,), jnp.int32),
            pltpu.VMEM((bpw, _GW), jnp.int32),
            pltpu.SemaphoreType.DMA,
        ],
    )
    def sc_gather(gb_hbm, idx_hbm, out_hbm, idx_v, rows_v, sem):
        wid = lax.axis_index("s") * info.num_cores + lax.axis_index("c")
        base = wid * bpw
        pltpu.sync_copy(idx_hbm.at[pl.ds(base, bpw)], idx_v)
        pltpu.async_copy(gb_hbm.at[idx_v], rows_v, sem).wait()
        pltpu.sync_copy(rows_v, out_hbm.at[pl.ds(base, bpw)])

    return sc_gather


def _merge_top2(av1, ap1, av2, ap2, bv1, bp1, bv2, bp2):
    swap = (bv1 > av1) | ((bv1 == av1) & (bp1 < ap1))
    w1v = jnp.where(swap, bv1, av1)
    w1p = jnp.where(swap, bp1, ap1)
    l1v = jnp.where(swap, av1, bv1)
    l1p = jnp.where(swap, ap1, bp1)
    c2v = jnp.where(swap, bv2, av2)
    c2p = jnp.where(swap, bp2, ap2)
    take2 = (c2v > l1v) | ((c2v == l1v) & (c2p < l1p))
    w2v = jnp.where(take2, c2v, l1v)
    w2p = jnp.where(take2, c2p, l1p)
    return w1v, w1p, w2v, w2p


def _scan_body(step_ref, x_ref,
               bm1_ref, bp1_ref, bm2_ref, bp2_ref, mrow_ref, logs_ref,
               ms_ref, ss_ref):
    vb = pl.program_id(0)
    masking = step_ref[0, 0] < _MINLEN
    sub8 = jax.lax.broadcasted_iota(jnp.int32, (_BEAM, _NR), 0)
    limit = _VOCAB - vb * _VB
    emask = masking & (vb == 0)

    @pl.when(vb == 0)
    def _init():
        ms_ref[...] = jnp.full((2 * _BEAM, _NR), _NEG, jnp.float32)
        ss_ref[...] = jnp.zeros((2 * _BEAM, _NR), jnp.float32)

    ma0 = ms_ref[pl.ds(0, _BEAM), :]
    mb0 = ms_ref[pl.ds(_BEAM, _BEAM), :]
    sa0 = ss_ref[pl.ds(0, _BEAM), :]
    sb0 = ss_ref[pl.ds(_BEAM, _BEAM), :]
    zneg = jnp.full((_BEAM, _NR), _NEG, jnp.float32)
    zi = jnp.zeros((_BEAM, _NR), jnp.int32)

    def one(s, m, ssum, v1, b1, v2, b2):
        x = x_ref[pl.ds(s * _BEAM, _BEAM), :]
        pos = s * _BEAM + sub8
        invalid = pos >= limit
        xs = jnp.where(invalid, _NEG, x)
        xc = jnp.where(invalid | (emask & (pos == _END)), _NEG, x)
        nm = jnp.maximum(m, xs)
        ssum = ssum * jnp.exp(m - nm) + jnp.exp(xs - nm)
        gt1 = xc > v1
        gt2 = xc > v2
        nv1 = jnp.maximum(xc, v1)
        nb1 = jnp.where(gt1, pos, b1)
        nv2 = jnp.where(gt1, v1, jnp.where(gt2, xc, v2))
        nb2 = jnp.where(gt1, b1, jnp.where(gt2, pos, b2))
        return nm, ssum, nv1, nb1, nv2, nb2

    def strip2(i, carry):
        ma, sa, mb, sb, v1, b1, v2, b2 = carry
        ma, sa, v1, b1, v2, b2 = one(2 * i, ma, sa, v1, b1, v2, b2)
        mb, sb, v1, b1, v2, b2 = one(2 * i + 1, mb, sb, v1, b1, v2, b2)
        return ma, sa, mb, sb, v1, b1, v2, b2

    ma, sa, mb, sb, v1, b1, v2, b2 = jax.lax.fori_loop(
        0, _NSTR // 2, strip2, (ma0, sa0, mb0, sb0, zneg, zi, zneg, zi))
    ms_ref[pl.ds(0, _BEAM), :] = ma
    ms_ref[pl.ds(_BEAM, _BEAM), :] = mb
    ss_ref[pl.ds(0, _BEAM), :] = sa
    ss_ref[pl.ds(_BEAM, _BEAM), :] = sb

    h = _BEAM
    while h > 1:
        h //= 2
        v1, b1, v2, b2 = _merge_top2(
            v1[0:h], b1[0:h], v2[0:h], b2[0:h],
            v1[h:2 * h], b1[h:2 * h], v2[h:2 * h], b2[h:2 * h])
    bm1_ref[...] = v1.reshape(1, 1, _NR)
    bp1_ref[...] = b1.reshape(1, 1, _NR)
    bm2_ref[...] = v2.reshape(1, 1, _NR)
    bp2_ref[...] = b2.reshape(1, 1, _NR)

    @pl.when(vb == _NB - 1)
    def _finalize():
        mm = ms_ref[...]
        sF = ss_ref[...]
        mf = mm
        for sh in (8, 4, 2, 1):
            mf = jnp.maximum(mf[0:sh], mf[sh:2 * sh])
        mf16 = jnp.broadcast_to(mf, (2 * _BEAM, _NR))
        corr = sF * jnp.exp(mm - mf16)
        for sh in (8, 4, 2, 1):
            corr = corr[0:sh] + corr[sh:2 * sh]
        mrow_ref[...] = mf.reshape(1, 1, _NR)
        logs_ref[...] = jnp.log(corr).reshape(1, 1, _NR)


def _extract_body(step_ref, pen_ref, c1_ref, q1_ref, c2_ref, q2_ref,
                  mr_ref, ls_ref, tlp_ref, lgt_ref,
                  sc_ref, lp_ref, tok_ref, row_ref, fin_ref,
                  lpsA_ref, gsA_ref, lpsB_ref, gsB_ref, scr_ref, sem):
    masking = step_ref[0, 0] < _MINLEN
    inv_pen = pen_ref[0, 0]

    lane98 = jax.lax.broadcasted_iota(jnp.int32, (_BEAM, _NB), 1)
    row8 = jax.lax.broadcasted_iota(jnp.int32, (_BEAM, _NB), 0)
    lane16 = jax.lax.broadcasted_iota(jnp.int32, (2 * _BEAM, _NB), 1)
    row16 = jax.lax.broadcasted_iota(jnp.int32, (2 * _BEAM, _NB), 0)
    row81 = jax.lax.broadcasted_iota(jnp.int32, (_BEAM, 1), 0)

    def setup(base, lps_x, gs_x):
        mrow = mr_ref[pl.ds(base, _BEAM), :]
        logS = ls_ref[pl.ds(base, _BEAM), :]
        tlp = tlp_ref[pl.ds(base, _BEAM), :]
        lp1 = ((c1_ref[pl.ds(base, _BEAM), :] - mrow) - logS) + tlp
        lp2 = ((c2_ref[pl.ds(base, _BEAM), :] - mrow) - logS) + tlp
        g1 = row8 * _VOCAB + lane98 * _VB + q1_ref[pl.ds(base, _BEAM), :]
        g2 = row8 * _VOCAB + lane98 * _VB + q2_ref[pl.ds(base, _BEAM), :]
        lps_x[...] = jnp.concatenate([lp1, lp2], axis=0)
        gs_x[...] = jnp.concatenate([g1, g2], axis=0)
        return mrow, logS, tlp

    def round_one(lps_x, gs_x):
        lp_all = lps_x[...]
        g_all = gs_x[...]
        w = jnp.max(lp_all)
        gw = jnp.min(jnp.where(lp_all == w, g_all, _BIGI))
        lps_x[...] = jnp.where(g_all == gw, jnp.float32(_NEG), lp_all)
        r = gw // _VOCAB
        pos = gw - r * _VOCAB
        vbw = pos // _VB
        return w, gw, r, vbw, r * _NB + vbw

    def rescan(b, r, vbw, picked, mrow, logS, tlp, lps_x, gs_x):
        rg = b * _BEAM + r
        s0 = jnp.minimum(vbw * _VB, _VOCAB - _VB)
        cp = pltpu.make_async_copy(
            lgt_ref.at[pl.ds(pl.multiple_of(s0, _BEAM), _VB), :],
            scr_ref, sem)
        cp.start()
        cp.wait()
        sub8 = jax.lax.broadcasted_iota(jnp.int32, (_BEAM, _NR), 0)
        lanes = jax.lax.broadcasted_iota(jnp.int32, (_BEAM, _NR), 1)

        def stepR(s, carry):
            vm, pm = carry
            x = scr_ref[pl.ds(s * _BEAM, _BEAM), :]
            p = s0 + s * _BEAM + sub8
            ok = (lanes == rg) & (p // _VB == vbw) & (p < _VOCAB)
            ok = ok & jnp.logical_not(masking & (p == _END))
            gv = r * _VOCAB + p
            for pg in picked:
                ok = ok & (gv != pg)
            xv = jnp.where(ok, x, jnp.float32(_NEG))
            upd = xv > vm
            return jnp.maximum(xv, vm), jnp.where(upd, p, pm)

        vm = jnp.full((_BEAM, _NR), _NEG, jnp.float32)
        pm = jnp.zeros((_BEAM, _NR), jnp.int32)
        vm, pm = jax.lax.fori_loop(0, _NSTR, stepR, (vm, pm))
        nv = jnp.max(vm)
        np_ = jnp.min(jnp.where(vm == nv, pm, _BIGI))
        m_r = jnp.sum(jnp.where(row81 == r, mrow, 0.0))
        ls_r = jnp.sum(jnp.where(row81 == r, logS, 0.0))
        tl_r = jnp.sum(jnp.where(row81 == r, tlp, 0.0))
        lp_new = ((nv - m_r) - ls_r) + tl_r
        g_new = r * _VOCAB + np_
        sel = (row16 == r) & (lane16 == vbw)
        lps_x[...] = jnp.where(sel, lp_new, lps_x[...])
        gs_x[...] = jnp.where(sel, g_new, gs_x[...])

    def pair(j, carry):
        bA = 2 * j
        bB = 2 * j + 1
        baseA = pl.multiple_of(bA * _BEAM, _BEAM)
        baseB = pl.multiple_of(bB * _BEAM, _BEAM)
        mrA, lsA, tlA = setup(baseA, lpsA_ref, gsA_ref)
        mrB, lsB, tlB = setup(baseB, lpsB_ref, gsB_ref)

        winA, pickA, winB, pickB = [], [], [], []
        wcolA = jnp.zeros((_BEAM, 1), jnp.float32)
        gcolA = jnp.zeros((_BEAM, 1), jnp.int32)
        wcolB = jnp.zeros((_BEAM, 1), jnp.float32)
        gcolB = jnp.zeros((_BEAM, 1), jnp.int32)

        for i in range(_BEAM):
            wA, gwA, rA, vbwA, bidA = round_one(lpsA_ref, gsA_ref)
            wB, gwB, rB, vbwB, bidB = round_one(lpsB_ref, gsB_ref)
            wcolA = wcolA + jnp.where(row81 == i, wA, 0.0)
            gcolA = gcolA + jnp.where(row81 == i, gwA, 0)
            wcolB = wcolB + jnp.where(row81 == i, wB, 0.0)
            gcolB = gcolB + jnp.where(row81 == i, gwB, 0)
            pickA.append(gwA)
            pickB.append(gwB)
            exA = jnp.bool_(False)
            for pb in winA:
                exA = jnp.logical_or(exA, pb == bidA)
            exB = jnp.bool_(False)
            for pb in winB:
                exB = jnp.logical_or(exB, pb == bidB)
            winA.append(bidA)
            winB.append(bidB)

            if i > 0:
                @pl.when(exA)
                def _rescanA(rA=rA, vbwA=vbwA, pk=tuple(pickA)):
                    rescan(bA, rA, vbwA, pk, mrA, lsA, tlA,
                           lpsA_ref, gsA_ref)

                @pl.when(exB)
                def _rescanB(rB=rB, vbwB=vbwB, pk=tuple(pickB)):
                    rescan(bB, rB, vbwB, pk, mrB, lsB, tlB,
                           lpsB_ref, gsB_ref)

        for base, wcol, gcol, b in ((baseA, wcolA, gcolA, bA),
                                    (baseB, wcolB, gcolB, bB)):
            rcol = gcol // _VOCAB
            tokcol = gcol - rcol * _VOCAB
            sc_ref[pl.ds(base, _BEAM), :] = wcol * inv_pen
            lp_ref[pl.ds(base, _BEAM), :] = wcol
            tok_ref[pl.ds(base, _BEAM), :] = tokcol
            row_ref[pl.ds(base, _BEAM), :] = rcol + b * _BEAM
            fin_ref[pl.ds(base, _BEAM), :] = (tokcol == _END).astype(jnp.int32)
        return carry

    jax.lax.fori_loop(0, _NR // (2 * _BEAM), pair, 0)


def kernel(logits, topk_log_probs, growing_beam, step):
    nrows = logits.shape[0]
    nb = nrows // _BEAM
    cur_len = growing_beam.shape[1]
    step2d = jnp.reshape(jnp.asarray(step, jnp.int32), (1, 1))
    length_penalty = ((5.0 + (jnp.asarray(step, jnp.int32) + 1)) / 6.0) ** 0.95
    invpen2d = jnp.reshape(
        (1.0 / length_penalty).astype(jnp.float32), (1, 1))
    tlp2d = jnp.reshape(topk_log_probs, (nrows, 1))

    lgt = jnp.swapaxes(logits, 0, 1)

    scan_out = pl.pallas_call(
        _scan_body,
        grid=(_NB,),
        in_specs=[
            pl.BlockSpec(memory_space=pltpu.SMEM),
            pl.BlockSpec((_VB, nrows), lambda vb: (vb, 0)),
        ],
        out_specs=[
            pl.BlockSpec((1, 1, nrows), lambda vb: (vb, 0, 0)),
            pl.BlockSpec((1, 1, nrows), lambda vb: (vb, 0, 0)),
            pl.BlockSpec((1, 1, nrows), lambda vb: (vb, 0, 0)),
            pl.BlockSpec((1, 1, nrows), lambda vb: (vb, 0, 0)),
            pl.BlockSpec((1, 1, nrows), lambda vb: (0, 0, 0)),
            pl.BlockSpec((1, 1, nrows), lambda vb: (0, 0, 0)),
        ],
        scratch_shapes=[
            pltpu.VMEM((2 * _BEAM, nrows), jnp.float32),
            pltpu.VMEM((2 * _BEAM, nrows), jnp.float32),
        ],
        out_shape=[
            jax.ShapeDtypeStruct((_NB, 1, nrows), jnp.float32),
            jax.ShapeDtypeStruct((_NB, 1, nrows), jnp.int32),
            jax.ShapeDtypeStruct((_NB, 1, nrows), jnp.float32),
            jax.ShapeDtypeStruct((_NB, 1, nrows), jnp.int32),
            jax.ShapeDtypeStruct((1, 1, nrows), jnp.float32),
            jax.ShapeDtypeStruct((1, 1, nrows), jnp.float32),
        ],
    )(step2d, lgt)
    bm1, bp1, bm2, bp2, mrow, logs = scan_out

    c1 = bm1.reshape(_NB, nrows).T
    q1 = bp1.reshape(_NB, nrows).T
    c2 = bm2.reshape(_NB, nrows).T
    q2 = bp2.reshape(_NB, nrows).T
    mrT = mrow.reshape(nrows, 1)
    lsT = logs.reshape(nrows, 1)

    out_shape = [
        jax.ShapeDtypeStruct((nrows, 1), jnp.float32),
        jax.ShapeDtypeStruct((nrows, 1), jnp.float32),
        jax.ShapeDtypeStruct((nrows, 1), jnp.int32),
        jax.ShapeDtypeStruct((nrows, 1), jnp.int32),
        jax.ShapeDtypeStruct((nrows, 1), jnp.int32),
    ]
    sc, lp, tok, rows, fin = pl.pallas_call(
        _extract_body,
        in_specs=[
            pl.BlockSpec(memory_space=pltpu.SMEM),
            pl.BlockSpec(memory_space=pltpu.SMEM),
            pl.BlockSpec(),
            pl.BlockSpec(),
            pl.BlockSpec(),
            pl.BlockSpec(),
            pl.BlockSpec(),
            pl.BlockSpec(),
            pl.BlockSpec(),
            pl.BlockSpec(memory_space=pl.ANY),
        ],
        out_specs=[
            pl.BlockSpec(),
            pl.BlockSpec(),
            pl.BlockSpec(),
            pl.BlockSpec(),
            pl.BlockSpec(),
        ],
        scratch_shapes=[
            pltpu.VMEM((2 * _BEAM, _NB), jnp.float32),
            pltpu.VMEM((2 * _BEAM, _NB), jnp.int32),
            pltpu.VMEM((2 * _BEAM, _NB), jnp.float32),
            pltpu.VMEM((2 * _BEAM, _NB), jnp.int32),
            pltpu.VMEM((_VB, nrows), jnp.float32),
            pltpu.SemaphoreType.DMA,
        ],
        out_shape=out_shape,
    )(step2d, invpen2d, c1, q1, c2, q2, mrT, lsT, tlp2d, lgt)

    rows_flat = rows.reshape(-1)
    gb_pad = jnp.pad(growing_beam, ((0, 0), (0, _GW - cur_len)))
    hist = _make_sc_gather(nrows)(gb_pad, rows_flat)
    nbm = jnp.concatenate([hist[:, :cur_len], tok], axis=1)

    return (sc.reshape(nb, _BEAM), lp.reshape(nb, _BEAM),
            tok.reshape(nb, _BEAM), rows_flat, nbm,
            (fin.reshape(nb, _BEAM) != 0))
